# baseline (speedup 1.0000x reference)
import functools

import jax
import jax.numpy as jnp
from jax import lax
from jax.experimental import pallas as pl
from jax.experimental.pallas import tpu as pltpu
from jax.experimental.pallas import tpu_sc as plsc

BATCH = 16384
EMBED = 32
LANES = 16
NUM_CORES = 2
NUM_SUBCORES = 16
NUM_WORKERS = NUM_CORES * NUM_SUBCORES
B_PER_W = BATCH // NUM_WORKERS
CHUNK = 128
NCHUNK = B_PER_W // CHUNK
BLOCKS = B_PER_W // LANES


def _tt_body(uid_hbm, mid_hbm, utab_hbm, itab_hbm, out_hbm,
             uidx_v, midx_v, urows_v, irows_v, out_v, sem):
    wid = lax.axis_index("s") * NUM_CORES + lax.axis_index("c")
    base = wid * B_PER_W

    for j in range(NCHUNK):
        pltpu.sync_copy(uid_hbm.at[pl.ds(base + j * CHUNK, CHUNK)], uidx_v.at[j])
        pltpu.sync_copy(mid_hbm.at[pl.ds(base + j * CHUNK, CHUNK)], midx_v.at[j])

    handles = []
    for j in range(NCHUNK):
        handles.append(pltpu.async_copy(
            utab_hbm.at[uidx_v.at[j]], urows_v.at[pl.ds(j * CHUNK, CHUNK)], sem))
        handles.append(pltpu.async_copy(
            itab_hbm.at[midx_v.at[j]], irows_v.at[pl.ds(j * CHUNK, CHUNK)], sem))
    for h in handles:
        h.wait()

    lanes = lax.iota(jnp.int32, LANES)

    def blk_body(b, carry):
        acc = jnp.zeros((LANES,), jnp.float32)
        for r in range(LANES):
            i = b * LANES + r
            u0 = urows_v[i, pl.ds(0, LANES)]
            u1 = urows_v[i, pl.ds(LANES, LANES)]
            v0 = irows_v[i, pl.ds(0, LANES)]
            v1 = irows_v[i, pl.ds(LANES, LANES)]
            s = jnp.sum(u0 * v0 + u1 * v1)
            acc = jnp.where(lanes == r, s, acc)
        out_v[pl.ds(b * LANES, LANES)] = 1.0 / (1.0 + jnp.exp(-acc))
        return carry

    lax.fori_loop(0, BLOCKS, blk_body, 0)

    pltpu.sync_copy(out_v, out_hbm.at[pl.ds(base, B_PER_W)])


_tt = functools.partial(
    pl.kernel,
    out_type=jax.ShapeDtypeStruct((BATCH,), jnp.float32),
    mesh=plsc.VectorSubcoreMesh(core_axis_name="c", subcore_axis_name="s"),
    scratch_types=[
        pltpu.VMEM((NCHUNK, CHUNK), jnp.int32),
        pltpu.VMEM((NCHUNK, CHUNK), jnp.int32),
        pltpu.VMEM((B_PER_W, EMBED), jnp.float32),
        pltpu.VMEM((B_PER_W, EMBED), jnp.float32),
        pltpu.VMEM((B_PER_W,), jnp.float32),
        pltpu.SemaphoreType.DMA,
    ],
    compiler_params=pltpu.CompilerParams(
        needs_layout_passes=False, use_tc_tiling_on_sc=False),
)(_tt_body)


def kernel(user_id, movie_id, user_table, item_table):
    return _tt(user_id.astype(jnp.int32), movie_id.astype(jnp.int32),
               user_table, item_table)
